# baseline (speedup 1.0000x reference)
import functools

import jax
import jax.numpy as jnp
from jax import lax
from jax.experimental import pallas as pl
from jax.experimental.pallas import tpu as pltpu
from jax.experimental.pallas import tpu_sc as plsc

N = 16384
D = 64
L = 16
NC = 2
NS = 16
NW = NC * NS
B = 128
NBUF = 4


def _rsqrt_nr(x):
    xh = x * 0.5
    i = plsc.bitcast(x, jnp.int32)
    i = jnp.int32(0x5F3759DF) - (i >> 1)
    y = plsc.bitcast(i, jnp.float32)
    y = y * (1.5 - xh * y * y)
    y = y * (1.5 - xh * y * y)
    y = y * (1.5 - xh * y * y)
    return y


def _make_sc_kernel(nblk):
    mesh = plsc.VectorSubcoreMesh(core_axis_name="c", subcore_axis_name="s")

    @functools.partial(
        pl.kernel,
        mesh=mesh,
        compiler_params=pltpu.CompilerParams(
            needs_layout_passes=False, use_tc_tiling_on_sc=False),
        out_type=jax.ShapeDtypeStruct((NW, L), jnp.float32),
        scratch_types=[
            pltpu.VMEM((nblk, B), jnp.int32),
            pltpu.VMEM((nblk, B), jnp.int32),
            pltpu.VMEM((nblk, B), jnp.float32),
            [pltpu.VMEM((B, D), jnp.float32) for _ in range(NBUF)],
            [pltpu.VMEM((B, D), jnp.float32) for _ in range(NBUF)],
            pltpu.VMEM((L,), jnp.float32),
            [pltpu.SemaphoreType.DMA for _ in range(NBUF)],
        ],
    )
    def sc_kernel(emb_hbm, ii_hbm, jj_hbm, sd_hbm, out_hbm,
                  ii_v, jj_v, sd_v, ri_v, rj_v, acc_v, gsem):
        wid = lax.axis_index("s") * NC + lax.axis_index("c")
        lane = lax.broadcasted_iota(jnp.int32, (L,), 0)

        pltpu.sync_copy(ii_hbm.at[wid], ii_v)
        pltpu.sync_copy(jj_hbm.at[wid], jj_v)
        pltpu.sync_copy(sd_hbm.at[wid], sd_v)

        def issue(b, t):
            pltpu.async_copy(emb_hbm.at[ii_v.at[t]], ri_v[b], gsem[b])
            pltpu.async_copy(emb_hbm.at[jj_v.at[t]], rj_v[b], gsem[b])

        for b in range(NBUF):
            issue(b, b)

        def group(g, acc):
            for b in range(NBUF):
                t = g * NBUF + b
                pltpu.make_async_copy(
                    emb_hbm.at[ii_v.at[t]], ri_v[b], gsem[b]).wait()
                pltpu.make_async_copy(
                    emb_hbm.at[jj_v.at[t]], rj_v[b], gsem[b]).wait()
                for q in range(B // L):
                    tot = jnp.zeros((L,), jnp.float32)
                    for r in range(L):
                        p = q * L + r
                        s = jnp.zeros((L,), jnp.float32)
                        for k in range(D // L):
                            vi = ri_v[b][p, pl.ds(k * L, L)]
                            vj = rj_v[b][p, pl.ds(k * L, L)]
                            df = vi - vj
                            s = s + df * df
                        tot = jnp.where(lane == r, jnp.sum(s), tot)
                    dist = tot * _rsqrt_nr(jnp.maximum(tot, 1e-30))
                    res = sd_v[t, pl.ds(q * L, L)] - dist
                    acc = acc + res * res
                tn = t + NBUF

                @pl.when(tn < nblk)
                def _():
                    issue(b, tn)

            return acc

        acc = lax.fori_loop(
            0, nblk // NBUF, group, jnp.zeros((L,), jnp.float32))
        acc_v[...] = acc
        pltpu.sync_copy(acc_v, out_hbm.at[wid])

    return sc_kernel


def kernel(emb, indices, small_dists):
    nnz = indices.shape[0]
    nblk = -(-nnz // (NW * B))
    nblk = -(-nblk // NBUF) * NBUF
    pad = NW * nblk * B - nnz
    ii = jnp.pad(indices[:, 0], (0, pad)).reshape(NW, nblk, B)
    jj = jnp.pad(indices[:, 1], (0, pad)).reshape(NW, nblk, B)
    sd = jnp.pad(small_dists, (0, pad)).reshape(NW, nblk, B)
    partial = _make_sc_kernel(nblk)(emb, ii, jj, sd)
    return jnp.sum(partial)


# speedup vs baseline: 2.2416x; 2.2416x over previous
import functools

import jax
import jax.numpy as jnp
from jax import lax
from jax.experimental import pallas as pl
from jax.experimental.pallas import tpu as pltpu
from jax.experimental.pallas import tpu_sc as plsc

N = 16384
D = 64
L = 16
NC = 2
NS = 16
NW = NC * NS
B = 128


def _rsqrt_nr(x):
    xh = x * 0.5
    i = plsc.bitcast(x, jnp.int32)
    i = jnp.int32(0x5F3759DF) - (i >> 1)
    y = plsc.bitcast(i, jnp.float32)
    y = y * (1.5 - xh * y * y)
    y = y * (1.5 - xh * y * y)
    y = y * (1.5 - xh * y * y)
    return y


def _make_sc_kernel(nblk):
    mesh = plsc.VectorSubcoreMesh(core_axis_name="c", subcore_axis_name="s")

    @functools.partial(
        pl.kernel,
        mesh=mesh,
        compiler_params=pltpu.CompilerParams(
            needs_layout_passes=False, use_tc_tiling_on_sc=False),
        out_type=jax.ShapeDtypeStruct((NW, L), jnp.float32),
        scratch_types=[
            pltpu.VMEM_SHARED((N, D), jnp.float32),
            pltpu.VMEM((nblk, B), jnp.int32),
            pltpu.VMEM((nblk, B), jnp.int32),
            pltpu.VMEM((nblk, B), jnp.float32),
            pltpu.VMEM((B, D), jnp.float32),
            pltpu.VMEM((B, D), jnp.float32),
            pltpu.VMEM((L,), jnp.float32),
            pltpu.SemaphoreType.DMA,
        ],
    )
    def sc_kernel(emb_hbm, ii_hbm, jj_hbm, sd_hbm, out_hbm,
                  emb_sh, ii_v, jj_v, sd_v, ri_v, rj_v, acc_v, sem):
        sid = lax.axis_index("s")
        wid = sid * NC + lax.axis_index("c")
        lane = lax.broadcasted_iota(jnp.int32, (L,), 0)

        rows = N // NS
        pltpu.sync_copy(emb_hbm.at[pl.ds(sid * rows, rows)],
                        emb_sh.at[pl.ds(sid * rows, rows)])
        pltpu.sync_copy(ii_hbm.at[wid], ii_v)
        pltpu.sync_copy(jj_hbm.at[wid], jj_v)
        pltpu.sync_copy(sd_hbm.at[wid], sd_v)
        plsc.subcore_barrier()

        def block(t, acc):
            pltpu.async_copy(emb_sh.at[ii_v.at[t]], ri_v, sem).wait()
            pltpu.async_copy(emb_sh.at[jj_v.at[t]], rj_v, sem).wait()
            for q in range(B // L):
                tot = jnp.zeros((L,), jnp.float32)
                for r in range(L):
                    p = q * L + r
                    s = jnp.zeros((L,), jnp.float32)
                    for k in range(D // L):
                        vi = ri_v[p, pl.ds(k * L, L)]
                        vj = rj_v[p, pl.ds(k * L, L)]
                        df = vi - vj
                        s = s + df * df
                    tot = jnp.where(lane == r, jnp.sum(s), tot)
                dist = tot * _rsqrt_nr(jnp.maximum(tot, 1e-30))
                res = sd_v[t, pl.ds(q * L, L)] - dist
                acc = acc + res * res
            return acc

        acc = lax.fori_loop(0, nblk, block, jnp.zeros((L,), jnp.float32))
        acc_v[...] = acc
        pltpu.sync_copy(acc_v, out_hbm.at[wid])

    return sc_kernel


def kernel(emb, indices, small_dists):
    nnz = indices.shape[0]
    nblk = -(-nnz // (NW * B))
    pad = NW * nblk * B - nnz
    ii = jnp.pad(indices[:, 0], (0, pad)).reshape(NW, nblk, B)
    jj = jnp.pad(indices[:, 1], (0, pad)).reshape(NW, nblk, B)
    sd = jnp.pad(small_dists, (0, pad)).reshape(NW, nblk, B)
    partial = _make_sc_kernel(nblk)(emb, ii, jj, sd)
    return jnp.sum(partial)


# speedup vs baseline: 4.1454x; 1.8493x over previous
import functools

import jax
import jax.numpy as jnp
from jax import lax
from jax.experimental import pallas as pl
from jax.experimental.pallas import tpu as pltpu
from jax.experimental.pallas import tpu_sc as plsc

N = 16384
D = 64
L = 16
NC = 2
NS = 16
NW = NC * NS
B = 128


_GATHER_DNUMS = lax.GatherDimensionNumbers(
    offset_dims=(), collapsed_slice_dims=(0,), start_index_map=(0,))


def _rot(x, k):
    idx = ((lax.broadcasted_iota(jnp.int32, (L,), 0) + k) % L).reshape(L, 1)
    return lax.gather(x, idx, _GATHER_DNUMS, slice_sizes=(1,),
                      mode=lax.GatherScatterMode.PROMISE_IN_BOUNDS)


def _rsqrt_nr(x):
    xh = x * 0.5
    i = plsc.bitcast(x, jnp.int32)
    i = jnp.int32(0x5F3759DF) - (i >> 1)
    y = plsc.bitcast(i, jnp.float32)
    y = y * (1.5 - xh * y * y)
    y = y * (1.5 - xh * y * y)
    y = y * (1.5 - xh * y * y)
    return y


def _make_sc_kernel(nblk):
    mesh = plsc.VectorSubcoreMesh(core_axis_name="c", subcore_axis_name="s")

    @functools.partial(
        pl.kernel,
        mesh=mesh,
        compiler_params=pltpu.CompilerParams(
            needs_layout_passes=False, use_tc_tiling_on_sc=False),
        out_type=jax.ShapeDtypeStruct((NW, L), jnp.float32),
        scratch_types=[
            pltpu.VMEM_SHARED((N, D), jnp.float32),
            pltpu.VMEM((nblk, B), jnp.int32),
            pltpu.VMEM((nblk, B), jnp.int32),
            pltpu.VMEM((nblk, B), jnp.float32),
            pltpu.VMEM((2, B, D), jnp.float32),
            pltpu.VMEM((2, B, D), jnp.float32),
            pltpu.VMEM((L,), jnp.float32),
            pltpu.SemaphoreType.DMA((2,)),
        ],
    )
    def sc_kernel(emb_hbm, ii_hbm, jj_hbm, sd_hbm, out_hbm,
                  emb_sh, ii_v, jj_v, sd_v, ri_v, rj_v, acc_v, sem):
        sid = lax.axis_index("s")
        wid = sid * NC + lax.axis_index("c")
        lane = lax.broadcasted_iota(jnp.int32, (L,), 0)

        rows = N // NS
        pltpu.sync_copy(emb_hbm.at[pl.ds(sid * rows, rows)],
                        emb_sh.at[pl.ds(sid * rows, rows)])
        pltpu.sync_copy(ii_hbm.at[wid], ii_v)
        pltpu.sync_copy(jj_hbm.at[wid], jj_v)
        pltpu.sync_copy(sd_hbm.at[wid], sd_v)
        plsc.subcore_barrier()

        def issue(b, t):
            pltpu.async_copy(emb_sh.at[ii_v.at[t]], ri_v.at[b], sem.at[b])
            pltpu.async_copy(emb_sh.at[jj_v.at[t]], rj_v.at[b], sem.at[b])

        issue(0, 0)
        issue(1, 1)

        def block(t, acc):
            b = lax.rem(t, 2)
            pltpu.make_async_copy(
                emb_sh.at[ii_v.at[t]], ri_v.at[b], sem.at[b]).wait()
            pltpu.make_async_copy(
                emb_sh.at[jj_v.at[t]], rj_v.at[b], sem.at[b]).wait()
            def qstep(q, acc):
                tot = jnp.zeros((L,), jnp.float32)
                for r in range(L):
                    p = q * L + r
                    s = jnp.zeros((L,), jnp.float32)
                    for k in range(D // L):
                        vi = ri_v[b, p, pl.ds(k * L, L)]
                        vj = rj_v[b, p, pl.ds(k * L, L)]
                        df = vi - vj
                        s = s + df * df
                    tot = jnp.where(lane == r, jnp.sum(s), tot)
                dist = tot * _rsqrt_nr(jnp.maximum(tot, 1e-30))
                res = sd_v[t, pl.ds(q * L, L)] - dist
                return acc + res * res

            acc = lax.fori_loop(0, B // L, qstep, acc)
            tn = t + 2

            @pl.when(tn < nblk)
            def _():
                issue(b, tn)

            return acc

        acc = lax.fori_loop(0, nblk, block, jnp.zeros((L,), jnp.float32))
        acc_v[...] = acc
        pltpu.sync_copy(acc_v, out_hbm.at[wid])

    return sc_kernel


def kernel(emb, indices, small_dists):
    nnz = indices.shape[0]
    nblk = -(-nnz // (NW * B))
    nblk = nblk + (nblk % 2)
    pad = NW * nblk * B - nnz
    ii = jnp.pad(indices[:, 0], (0, pad)).reshape(NW, nblk, B)
    jj = jnp.pad(indices[:, 1], (0, pad)).reshape(NW, nblk, B)
    sd = jnp.pad(small_dists, (0, pad)).reshape(NW, nblk, B)
    partial = _make_sc_kernel(nblk)(emb, ii, jj, sd)
    return jnp.sum(partial)


# speedup vs baseline: 6.0719x; 1.4647x over previous
import functools

import jax
import jax.numpy as jnp
from jax import lax
from jax.experimental import pallas as pl
from jax.experimental.pallas import tpu as pltpu
from jax.experimental.pallas import tpu_sc as plsc

N = 16384
D = 64
L = 16
NC = 2
NS = 16
NW = NC * NS
B = 128


_GATHER_DNUMS = lax.GatherDimensionNumbers(
    offset_dims=(), collapsed_slice_dims=(0,), start_index_map=(0,))


def _rot(x, k):
    idx = ((lax.broadcasted_iota(jnp.int32, (L,), 0) + k) % L).reshape(L, 1)
    return lax.gather(x, idx, _GATHER_DNUMS, slice_sizes=(1,),
                      mode=lax.GatherScatterMode.PROMISE_IN_BOUNDS)


def _rsqrt_nr(x):
    xh = x * 0.5
    i = plsc.bitcast(x, jnp.int32)
    i = jnp.int32(0x5F3759DF) - (i >> 1)
    y = plsc.bitcast(i, jnp.float32)
    y = y * (1.5 - xh * y * y)
    y = y * (1.5 - xh * y * y)
    y = y * (1.5 - xh * y * y)
    return y


def _make_sc_kernel(nblk):
    mesh = plsc.VectorSubcoreMesh(core_axis_name="c", subcore_axis_name="s")

    @functools.partial(
        pl.kernel,
        mesh=mesh,
        compiler_params=pltpu.CompilerParams(
            needs_layout_passes=False, use_tc_tiling_on_sc=False),
        out_type=jax.ShapeDtypeStruct((NW, L), jnp.float32),
        scratch_types=[
            pltpu.VMEM_SHARED((N, D), jnp.bfloat16),
            pltpu.VMEM((nblk, B), jnp.int32),
            pltpu.VMEM((nblk, B), jnp.int32),
            pltpu.VMEM((nblk, B), jnp.float32),
            pltpu.VMEM((2, B, D), jnp.bfloat16),
            pltpu.VMEM((2, B, D), jnp.bfloat16),
            pltpu.VMEM((L,), jnp.float32),
            pltpu.SemaphoreType.DMA((2,)),
        ],
    )
    def sc_kernel(emb_hbm, ii_hbm, jj_hbm, sd_hbm, out_hbm,
                  emb_sh, ii_v, jj_v, sd_v, ri_v, rj_v, acc_v, sem):
        sid = lax.axis_index("s")
        wid = sid * NC + lax.axis_index("c")
        lane = lax.broadcasted_iota(jnp.int32, (L,), 0)

        rows = N // NS
        pltpu.sync_copy(emb_hbm.at[pl.ds(sid * rows, rows)],
                        emb_sh.at[pl.ds(sid * rows, rows)])
        pltpu.sync_copy(ii_hbm.at[wid], ii_v)
        pltpu.sync_copy(jj_hbm.at[wid], jj_v)
        pltpu.sync_copy(sd_hbm.at[wid], sd_v)
        plsc.subcore_barrier()

        def issue(b, t):
            pltpu.async_copy(emb_sh.at[ii_v.at[t]], ri_v.at[b], sem.at[b])
            pltpu.async_copy(emb_sh.at[jj_v.at[t]], rj_v.at[b], sem.at[b])

        issue(0, 0)
        issue(1, 1)

        def block(t, acc):
            b = lax.rem(t, 2)
            pltpu.make_async_copy(
                emb_sh.at[ii_v.at[t]], ri_v.at[b], sem.at[b]).wait()
            pltpu.make_async_copy(
                emb_sh.at[jj_v.at[t]], rj_v.at[b], sem.at[b]).wait()
            def qstep(q, acc):
                tot = jnp.zeros((L,), jnp.float32)
                for r in range(L):
                    p = q * L + r
                    s = jnp.zeros((L,), jnp.float32)
                    for k in range(D // (2 * L)):
                        vi = ri_v[b, p, pl.ds(k * 2 * L, 2 * L)]
                        vj = rj_v[b, p, pl.ds(k * 2 * L, 2 * L)]
                        df = vi - vj
                        d0, d1 = plsc.unpack(
                            df, format=plsc.PackFormat.INTERLEAVED)
                        s = s + d0 * d0 + d1 * d1
                    tot = jnp.where(lane == r, jnp.sum(s), tot)
                dist = tot * _rsqrt_nr(jnp.maximum(tot, 1e-30))
                res = sd_v[t, pl.ds(q * L, L)] - dist
                return acc + res * res

            acc = lax.fori_loop(0, B // L, qstep, acc)
            tn = t + 2

            @pl.when(tn < nblk)
            def _():
                issue(b, tn)

            return acc

        acc = lax.fori_loop(0, nblk, block, jnp.zeros((L,), jnp.float32))
        acc_v[...] = acc
        pltpu.sync_copy(acc_v, out_hbm.at[wid])

    return sc_kernel


def kernel(emb, indices, small_dists):
    nnz = indices.shape[0]
    nblk = -(-nnz // (NW * B))
    nblk = nblk + (nblk % 2)
    pad = NW * nblk * B - nnz
    ii = jnp.pad(indices[:, 0], (0, pad)).reshape(NW, nblk, B)
    jj = jnp.pad(indices[:, 1], (0, pad)).reshape(NW, nblk, B)
    sd = jnp.pad(small_dists, (0, pad)).reshape(NW, nblk, B)
    partial = _make_sc_kernel(nblk)(emb.astype(jnp.bfloat16), ii, jj, sd)
    return jnp.sum(partial)


# speedup vs baseline: 6.2145x; 1.0235x over previous
import functools

import jax
import jax.numpy as jnp
from jax import lax
from jax.experimental import pallas as pl
from jax.experimental.pallas import tpu as pltpu
from jax.experimental.pallas import tpu_sc as plsc

N = 16384
NBITS = 14
D = 64
L = 16
NC = 2
NS = 16
NW = NC * NS
B = 128


def _rsqrt_nr(x):
    xh = x * 0.5
    i = plsc.bitcast(x, jnp.int32)
    i = jnp.int32(0x5F3759DF) - (i >> 1)
    y = plsc.bitcast(i, jnp.float32)
    y = y * (1.5 - xh * y * y)
    y = y * (1.5 - xh * y * y)
    y = y * (1.5 - xh * y * y)
    return y


def _make_sc_kernel(nblk):
    mesh = plsc.VectorSubcoreMesh(core_axis_name="c", subcore_axis_name="s")

    @functools.partial(
        pl.kernel,
        mesh=mesh,
        compiler_params=pltpu.CompilerParams(
            needs_layout_passes=False, use_tc_tiling_on_sc=False),
        out_type=jax.ShapeDtypeStruct((NW, L), jnp.float32),
        scratch_types=[
            pltpu.VMEM_SHARED((N, D), jnp.bfloat16),
            pltpu.VMEM((nblk, B), jnp.int32),
            pltpu.VMEM((nblk, B), jnp.float32),
            pltpu.VMEM((2, B), jnp.int32),
            pltpu.VMEM((2, B), jnp.int32),
            pltpu.VMEM((2, B, D), jnp.bfloat16),
            pltpu.VMEM((2, B, D), jnp.bfloat16),
            pltpu.VMEM((L,), jnp.float32),
            pltpu.SemaphoreType.DMA((2,)),
        ],
    )
    def sc_kernel(emb_hbm, pk_hbm, sd_hbm, out_hbm,
                  emb_sh, pk_v, sd_v, ii_v, jj_v, ri_v, rj_v, acc_v, sem):
        sid = lax.axis_index("s")
        wid = sid * NC + lax.axis_index("c")
        lane = lax.broadcasted_iota(jnp.int32, (L,), 0)

        rows = N // NS
        pltpu.sync_copy(emb_hbm.at[pl.ds(sid * rows, rows)],
                        emb_sh.at[pl.ds(sid * rows, rows)])
        pltpu.sync_copy(pk_hbm.at[wid], pk_v)
        pltpu.sync_copy(sd_hbm.at[wid], sd_v)
        plsc.subcore_barrier()

        def issue(b, t):
            for c in range(B // L):
                x = pk_v[t, pl.ds(c * L, L)]
                ii_v[b, pl.ds(c * L, L)] = x & (N - 1)
                jj_v[b, pl.ds(c * L, L)] = lax.shift_right_logical(x, NBITS)
            pltpu.async_copy(emb_sh.at[ii_v.at[b]], ri_v.at[b], sem.at[b])
            pltpu.async_copy(emb_sh.at[jj_v.at[b]], rj_v.at[b], sem.at[b])

        issue(0, 0)
        issue(1, 1)

        def block(t, acc):
            b = lax.rem(t, 2)
            pltpu.make_async_copy(
                emb_sh.at[ii_v.at[b]], ri_v.at[b], sem.at[b]).wait()
            pltpu.make_async_copy(
                emb_sh.at[jj_v.at[b]], rj_v.at[b], sem.at[b]).wait()

            def qstep(q, acc):
                tot = jnp.zeros((L,), jnp.float32)
                for r in range(L):
                    p = q * L + r
                    s = jnp.zeros((L,), jnp.float32)
                    for k in range(D // (2 * L)):
                        vi = ri_v[b, p, pl.ds(k * 2 * L, 2 * L)]
                        vj = rj_v[b, p, pl.ds(k * 2 * L, 2 * L)]
                        df = vi - vj
                        d0, d1 = plsc.unpack(
                            df, format=plsc.PackFormat.INTERLEAVED)
                        s = s + d0 * d0 + d1 * d1
                    tot = jnp.where(lane == r, jnp.sum(s), tot)
                dist = tot * _rsqrt_nr(jnp.maximum(tot, 1e-30))
                res = sd_v[t, pl.ds(q * L, L)] - dist
                return acc + res * res

            acc = lax.fori_loop(0, B // L, qstep, acc)
            tn = t + 2

            @pl.when(tn < nblk)
            def _():
                issue(b, tn)

            return acc

        acc = lax.fori_loop(0, nblk, block, jnp.zeros((L,), jnp.float32))
        acc_v[...] = acc
        pltpu.sync_copy(acc_v, out_hbm.at[wid])

    return sc_kernel


def kernel(emb, indices, small_dists):
    nnz = indices.shape[0]
    nblk = -(-nnz // (NW * B))
    nblk = nblk + (nblk % 2)
    pad = NW * nblk * B - nnz
    packed = indices[:, 0] + indices[:, 1] * N
    pk = jnp.pad(packed, (0, pad)).reshape(NW, nblk, B)
    sd = jnp.pad(small_dists, (0, pad)).reshape(NW, nblk, B)
    partial = _make_sc_kernel(nblk)(emb.astype(jnp.bfloat16), pk, sd)
    return jnp.sum(partial)
